# baseline (speedup 1.0000x reference)
import jax
import jax.numpy as jnp
from jax import lax
from jax.experimental import pallas as pl
from jax.experimental.pallas import tpu as pltpu
from jax.experimental.pallas import tpu_sc as plsc

N = 100000
D = 128
K = 64
BLK = 1024
GRID = (N + BLK - 1) // BLK
CAND = GRID * K
NSUB = 32
CH_PER = D // NSUB

NEG = float("-inf")


def _ce_lane(x, d, take_min, low):
    a = pltpu.roll(x, D - d, 1)
    b = pltpu.roll(x, d, 1)
    xp = jnp.where(low, a, b)
    return jnp.where(take_min, jnp.minimum(x, xp), jnp.maximum(x, xp))


def _ce_sub(x, d, take_max, low):
    a = pltpu.roll(x, K - d, 0)
    b = pltpu.roll(x, d, 0)
    xp = jnp.where(low, a, b)
    return jnp.where(take_max, jnp.maximum(x, xp), jnp.minimum(x, xp))


def _sort_rows_asc(x, lane):
    kk = 2
    while kk <= D:
        desc = (lane & kk) != 0
        d = kk // 2
        while d >= 1:
            low = (lane & d) == 0
            take_min = jnp.logical_xor(low, desc)
            x = _ce_lane(x, d, take_min, low)
            d //= 2
        kk *= 2
    return x


def _sort64(x, row, desc):
    kk = 2
    while kk <= K:
        blk = (row & kk) != 0
        d = kk // 2
        while d >= 1:
            low = (row & d) == 0
            m = jnp.logical_xor(low, blk)
            take_max = m if desc else jnp.logical_not(m)
            x = _ce_sub(x, d, take_max, low)
            d //= 2
        kk *= 2
    return x


def _clean64(c, row, desc):
    d = K // 2
    while d >= 1:
        low = (row & d) == 0
        take_max = low if desc else jnp.logical_not(low)
        c = _ce_sub(c, d, take_max, low)
        d //= 2
    return c


def _merge64(a_desc, b_asc, row, desc):
    return _clean64(jnp.maximum(a_desc, b_asc), row, desc)


def _block_top64(tiles, row, desc):
    if len(tiles) == 1:
        return _sort64(tiles[0], row, desc)
    h = len(tiles) // 2
    a = _block_top64(tiles[:h], row, True)
    b = _block_top64(tiles[h:], row, False)
    return _merge64(a, b, row, desc)


def _tc_body(x_ref, o_ref):
    i = pl.program_id(0)
    x = x_ref[...]
    rowg = jax.lax.broadcasted_iota(jnp.int32, (BLK, 1), 0) + i * BLK
    x = jnp.where(rowg < N, x, NEG)
    lane = jax.lax.broadcasted_iota(jnp.int32, (1, D), 1)
    x = _sort_rows_asc(x, lane)

    row = jax.lax.broadcasted_iota(jnp.int32, (K, 1), 0)
    tiles = [x[t * K:(t + 1) * K, :] for t in range(BLK // K)]
    o_ref[...] = _block_top64(tiles, row, desc=True)


def _run_tc(feat, interpret=False):
    return pl.pallas_call(
        _tc_body,
        grid=(GRID,),
        in_specs=[pl.BlockSpec((BLK, D), lambda i: (i, 0))],
        out_specs=pl.BlockSpec((K, D), lambda i: (i, 0)),
        out_shape=jax.ShapeDtypeStruct((CAND, D), jnp.float32),
        compiler_params=pltpu.CompilerParams(
            dimension_semantics=("parallel",)),
        interpret=interpret,
    )(feat)


def _ce16_desc(z, d):
    i16 = lax.iota(jnp.int32, 16)
    p = jnp.take_along_axis(z, i16 ^ d, axis=0, mode="promise_in_bounds")
    low = (i16 & d) == 0
    return jnp.where(low, jnp.maximum(z, p), jnp.minimum(z, p))


def _sc_merge_desc(best, run):
    rev = [lax.rev(run[3 - t], (0,)) for t in range(4)]
    c = [jnp.maximum(best[t], rev[t]) for t in range(4)]
    y0, y2 = jnp.maximum(c[0], c[2]), jnp.minimum(c[0], c[2])
    y1, y3 = jnp.maximum(c[1], c[3]), jnp.minimum(c[1], c[3])
    z0, z1 = jnp.maximum(y0, y1), jnp.minimum(y0, y1)
    z2, z3 = jnp.maximum(y2, y3), jnp.minimum(y2, y3)
    out = []
    for z in (z0, z1, z2, z3):
        for d in (8, 4, 2, 1):
            z = _ce16_desc(z, d)
        out.append(z)
    return tuple(out)


def _sc_fold(cand_hbm, out_hbm, colbuf, outbuf):
    wid = lax.axis_index("s") * 2 + lax.axis_index("c")
    bc = wid * CH_PER
    pltpu.sync_copy(cand_hbm.at[pl.ds(bc, CH_PER)], colbuf)
    for j in range(CH_PER):
        best = tuple(colbuf[j, 16 * t:16 * (t + 1)] for t in range(4))

        def body(r, b, j=j):
            run = tuple(colbuf[j, pl.ds(r * K + 16 * t, 16)]
                        for t in range(4))
            return _sc_merge_desc(b, run)

        best = lax.fori_loop(1, GRID, body, best)
        for t in range(4):
            outbuf[j, 16 * t:16 * (t + 1)] = best[t]
    pltpu.sync_copy(outbuf, out_hbm.at[pl.ds(bc, CH_PER)])


def _run_sc(cand_t):
    mesh = plsc.VectorSubcoreMesh(core_axis_name="c", subcore_axis_name="s")
    f = pl.kernel(
        _sc_fold,
        out_type=jax.ShapeDtypeStruct((D, K), jnp.float32),
        mesh=mesh,
        scratch_types=[
            pltpu.VMEM((CH_PER, CAND), jnp.float32),
            pltpu.VMEM((CH_PER, K), jnp.float32),
        ],
    )
    return f(cand_t)


@jax.jit
def kernel(feat):
    cand = _run_tc(feat)
    scout = _run_sc(cand.T)
    return scout.T.reshape(K * D)
